# Initial kernel scaffold; baseline (speedup 1.0000x reference)
import jax
import jax.numpy as jnp
from jax.experimental import pallas as pl


def kernel(x, base_res, lora_results, W, b):
    raise NotImplementedError("write your pallas kernel here")


# speedup vs baseline: 4.6038x; 4.6038x over previous
import jax
import jax.numpy as jnp
from jax.experimental import pallas as pl
from jax.experimental.pallas import tpu as pltpu

LANES = 128
TOPK = 2


def _combine_kernel(pooled_ref, w_ref, b_ref, lora_ref, base_ref, out_ref, p_ref):
    t = pl.program_id(1)
    E = w_ref.shape[0]
    G = LANES // E

    @pl.when(t == 0)
    def _build_p():
        pooled = pooled_ref[0]
        gate = jax.lax.dot_general(
            pooled, w_ref[...], (((1,), (1,)), ((), ())),
            preferred_element_type=jnp.float32,
            precision=jax.lax.Precision.HIGHEST,
        ) + b_ref[...]
        lane = jax.lax.broadcasted_iota(jnp.int32, (1, E), 1)
        gs = [jnp.sum(jnp.where(lane == e, gate, 0.0)) for e in range(E)]
        li = jax.lax.broadcasted_iota(jnp.int32, (LANES, G), 0)
        gi = jax.lax.broadcasted_iota(jnp.int32, (LANES, G), 1)
        p = jnp.zeros((LANES, G), jnp.float32)
        for i in range(E):
            rank = jnp.int32(0)
            for j in range(E):
                if j == i:
                    continue
                beats = (gs[j] >= gs[i]) if j < i else (gs[j] > gs[i])
                rank = rank + beats.astype(jnp.int32)
            m_i = (rank < TOPK).astype(jnp.float32)
            sel = ((li % E == i) & (li // E == gi)).astype(jnp.float32)
            p = p + m_i * sel
        p_ref[...] = p

    acc = jax.lax.dot_general(
        lora_ref[0], p_ref[...], (((1,), (0,)), ((), ())),
        preferred_element_type=jnp.float32,
        precision=jax.lax.Precision.HIGHEST,
    )
    out_ref[0] = base_ref[0] + acc


def kernel(x, base_res, lora_results, W, b):
    B, S, D, E = lora_results.shape
    G = LANES // E
    ROWS = S * D * E // LANES
    M = 2048

    pooled = x[:, -1:, :]
    lora2 = lora_results.reshape(B, ROWS, LANES)
    base2 = base_res.reshape(B, ROWS, G)
    b2 = b.reshape(1, E)

    out2 = pl.pallas_call(
        _combine_kernel,
        grid=(B, ROWS // M),
        in_specs=[
            pl.BlockSpec((1, 1, D), lambda bi, ti: (bi, 0, 0)),
            pl.BlockSpec((E, D), lambda bi, ti: (0, 0)),
            pl.BlockSpec((1, E), lambda bi, ti: (0, 0)),
            pl.BlockSpec((1, M, LANES), lambda bi, ti: (bi, ti, 0)),
            pl.BlockSpec((1, M, G), lambda bi, ti: (bi, ti, 0)),
        ],
        out_specs=pl.BlockSpec((1, M, G), lambda bi, ti: (bi, ti, 0)),
        out_shape=jax.ShapeDtypeStruct((B, ROWS, G), jnp.float32),
        scratch_shapes=[pltpu.VMEM((LANES, G), jnp.float32)],
    )(pooled, W, b2, lora2, base2)
    return out2.reshape(B, S, D)


# speedup vs baseline: 30.1915x; 6.5580x over previous
import jax
import jax.numpy as jnp
from jax.experimental import pallas as pl
from jax.experimental.pallas import tpu as pltpu

LANES = 128
TOPK = 2


def _combine_kernel(pooled_ref, w_ref, b_ref, lora_ref, base_ref, out_ref, p_ref):
    t = pl.program_id(1)
    c = pl.program_id(2)
    E = w_ref.shape[0]
    K = E * LANES

    @pl.when((t == 0) & (c == 0))
    def _build_p():
        pooled = pooled_ref[0]
        gate = jax.lax.dot_general(
            pooled, w_ref[...], (((1,), (1,)), ((), ())),
            preferred_element_type=jnp.float32,
            precision=jax.lax.Precision.HIGHEST,
        ) + b_ref[...]
        lane = jax.lax.broadcasted_iota(jnp.int32, (1, E), 1)
        gs = [jnp.sum(jnp.where(lane == e, gate, 0.0)) for e in range(E)]
        li = jax.lax.broadcasted_iota(jnp.int32, (K, LANES), 0)
        oi = jax.lax.broadcasted_iota(jnp.int32, (K, LANES), 1)
        p = jnp.zeros((K, LANES), jnp.float32)
        for i in range(E):
            rank = jnp.int32(0)
            for j in range(E):
                if j == i:
                    continue
                beats = (gs[j] >= gs[i]) if j < i else (gs[j] > gs[i])
                rank = rank + beats.astype(jnp.int32)
            m_i = (rank < TOPK).astype(jnp.float32)
            sel = ((li % E == i) & (oi == li // E)).astype(jnp.float32)
            p = p + m_i * sel
        p_ref[...] = p

    acc = jax.lax.dot_general(
        lora_ref[0], p_ref[...], (((1,), (0,)), ((), ())),
        preferred_element_type=jnp.float32,
    )
    out_ref[0] = base_ref[0] + acc


def kernel(x, base_res, lora_results, W, b):
    B, S, D, E = lora_results.shape
    K = E * LANES
    NC = D // LANES
    TS = 512

    pooled = x[:, -1:, :]
    lora3 = lora_results.reshape(B, S, D * E)
    b2 = b.reshape(1, E)

    out = pl.pallas_call(
        _combine_kernel,
        grid=(B, S // TS, NC),
        in_specs=[
            pl.BlockSpec((1, 1, D), lambda bi, ti, ci: (bi, 0, 0)),
            pl.BlockSpec((E, D), lambda bi, ti, ci: (0, 0)),
            pl.BlockSpec((1, E), lambda bi, ti, ci: (0, 0)),
            pl.BlockSpec((1, TS, K), lambda bi, ti, ci: (bi, ti, ci)),
            pl.BlockSpec((1, TS, LANES), lambda bi, ti, ci: (bi, ti, ci)),
        ],
        out_specs=pl.BlockSpec((1, TS, LANES), lambda bi, ti, ci: (bi, ti, ci)),
        out_shape=jax.ShapeDtypeStruct((B, S, D), jnp.float32),
        scratch_shapes=[pltpu.VMEM((K, LANES), jnp.float32)],
    )(pooled, W, b2, lora3, base_res)
    return out


# speedup vs baseline: 389.2825x; 12.8938x over previous
import jax
import jax.numpy as jnp
from jax.experimental import pallas as pl
from jax.experimental.pallas import tpu as pltpu

TOPK = 2
TS = 512


def _gate_kernel(pooled_ref, w_ref, b_ref, idx_ref):
    B, E = pooled_ref.shape[0], w_ref.shape[0]
    g = jax.lax.dot_general(
        pooled_ref[...], w_ref[...], (((1,), (1,)), ((), ())),
        preferred_element_type=jnp.float32,
        precision=jax.lax.Precision.HIGHEST,
    ) + b_ref[...]
    lane = jax.lax.broadcasted_iota(jnp.int32, (B, E), 1)
    rank = jnp.zeros((B, E), jnp.int32)
    for j in range(E):
        gj = g[:, j:j + 1]
        beats = (gj > g) | ((gj == g) & (j < lane))
        rank = rank + beats.astype(jnp.int32)
    cols = [
        jnp.sum(jnp.where(rank == k, lane, 0), axis=1, keepdims=True)
        for k in range(TOPK)
    ]
    idx_ref[0] = jnp.concatenate(cols, axis=1)


def _combine_kernel(idx_ref, lora_ref, base_ref, out_ref, buf, sem):
    g = pl.program_id(0)
    total = pl.num_programs(0)
    S = lora_ref.shape[1]
    nt = S // TS

    def start(gg, slot):
        b = gg // nt
        t = gg % nt
        for k in range(TOPK):
            e = idx_ref[b, k]
            pltpu.make_async_copy(
                lora_ref.at[b, pl.ds(t * TS, TS), e, :],
                buf.at[slot, k],
                sem.at[slot, k],
            ).start()

    p = jax.lax.rem(g, 2)

    @pl.when(g == 0)
    def _first():
        start(g, p)

    @pl.when(g + 1 < total)
    def _prefetch_next():
        start(g + 1, 1 - p)

    def wait(gg, slot):
        b = gg // nt
        t = gg % nt
        for k in range(TOPK):
            e = idx_ref[b, k]
            pltpu.make_async_copy(
                lora_ref.at[b, pl.ds(t * TS, TS), e, :],
                buf.at[slot, k],
                sem.at[slot, k],
            ).wait()

    wait(g, p)
    out_ref[0] = base_ref[0] + (buf[p, 0] + buf[p, 1])


def kernel(x, base_res, lora_results, W, b):
    B, S, D, E = lora_results.shape
    nt = S // TS

    pooled = x[:, -1, :]
    lora_t = lora_results.transpose(0, 1, 3, 2)
    b2 = b.reshape(1, E)

    idx3 = pl.pallas_call(
        _gate_kernel,
        grid=(1,),
        in_specs=[
            pl.BlockSpec((B, D), lambda i: (0, 0)),
            pl.BlockSpec((E, D), lambda i: (0, 0)),
            pl.BlockSpec((1, E), lambda i: (0, 0)),
        ],
        out_specs=pl.BlockSpec((1, B, TOPK), lambda i: (0, 0, 0)),
        out_shape=jax.ShapeDtypeStruct((1, B, TOPK), jnp.int32),
    )(pooled, W, b2)
    idx = idx3.reshape(B, TOPK)

    out = pl.pallas_call(
        _combine_kernel,
        grid=(B * nt,),
        in_specs=[
            pl.BlockSpec(memory_space=pltpu.MemorySpace.SMEM),
            pl.BlockSpec(memory_space=pltpu.MemorySpace.HBM),
            pl.BlockSpec((1, TS, D), lambda g: (g // nt, g % nt, 0)),
        ],
        out_specs=pl.BlockSpec((1, TS, D), lambda g: (g // nt, g % nt, 0)),
        out_shape=jax.ShapeDtypeStruct((B, S, D), jnp.float32),
        scratch_shapes=[
            pltpu.VMEM((2, TOPK, TS, D), jnp.float32),
            pltpu.SemaphoreType.DMA((2, TOPK)),
        ],
    )(idx, lora_t, base_res)
    return out


# speedup vs baseline: 401.6223x; 1.0317x over previous
import jax
import jax.numpy as jnp
from jax.experimental import pallas as pl
from jax.experimental.pallas import tpu as pltpu

TOPK = 2
TS = 512


def _combine_kernel(pooled_ref, w_ref, b_ref, lora_ref, base_ref, out_ref,
                    idx_ref, buf, sem):
    g = pl.program_id(0)
    total = pl.num_programs(0)
    B, E = pooled_ref.shape[0], w_ref.shape[0]
    S = lora_ref.shape[1]
    nt = S // TS

    @pl.when(g == 0)
    def _gate():
        gate = jax.lax.dot_general(
            pooled_ref[...], w_ref[...], (((1,), (1,)), ((), ())),
            preferred_element_type=jnp.float32,
            precision=jax.lax.Precision.HIGHEST,
        ) + b_ref[...]
        lane = jax.lax.broadcasted_iota(jnp.int32, (B, E), 1)
        rank = jnp.zeros((B, E), jnp.int32)
        for j in range(E):
            gj = gate[:, j:j + 1]
            beats = (gj > gate) | ((gj == gate) & (j < lane))
            rank = rank + beats.astype(jnp.int32)
        for bb in range(B):
            for k in range(TOPK):
                idx_ref[bb, k] = jnp.sum(
                    jnp.where(rank[bb:bb + 1, :] == k, lane[:1], 0))

    def start(gg, slot):
        b = gg // nt
        t = gg % nt
        for k in range(TOPK):
            e = idx_ref[b, k]
            pltpu.make_async_copy(
                lora_ref.at[b, pl.ds(t * TS, TS), e, :],
                buf.at[slot, k],
                sem.at[slot, k],
            ).start()

    p = jax.lax.rem(g, 2)

    @pl.when(g == 0)
    def _first():
        start(g, p)

    @pl.when(g + 1 < total)
    def _prefetch_next():
        start(g + 1, 1 - p)

    def wait(gg, slot):
        b = gg // nt
        t = gg % nt
        for k in range(TOPK):
            e = idx_ref[b, k]
            pltpu.make_async_copy(
                lora_ref.at[b, pl.ds(t * TS, TS), e, :],
                buf.at[slot, k],
                sem.at[slot, k],
            ).wait()

    wait(g, p)
    out_ref[0] = base_ref[0] + (buf[p, 0] + buf[p, 1])


def kernel(x, base_res, lora_results, W, b):
    B, S, D, E = lora_results.shape
    nt = S // TS

    pooled = x[:, -1, :]
    lora_t = lora_results.transpose(0, 1, 3, 2)
    b2 = b.reshape(1, E)

    out = pl.pallas_call(
        _combine_kernel,
        grid=(B * nt,),
        in_specs=[
            pl.BlockSpec((B, D), lambda g: (0, 0)),
            pl.BlockSpec((E, D), lambda g: (0, 0)),
            pl.BlockSpec((1, E), lambda g: (0, 0)),
            pl.BlockSpec(memory_space=pltpu.MemorySpace.HBM),
            pl.BlockSpec((1, TS, D), lambda g: (g // nt, g % nt, 0)),
        ],
        out_specs=pl.BlockSpec((1, TS, D), lambda g: (g // nt, g % nt, 0)),
        out_shape=jax.ShapeDtypeStruct((B, S, D), jnp.float32),
        scratch_shapes=[
            pltpu.SMEM((B, TOPK), jnp.int32),
            pltpu.VMEM((2, TOPK, TS, D), jnp.float32),
            pltpu.SemaphoreType.DMA((2, TOPK)),
        ],
    )(pooled, W, b2, lora_t, base_res)
    return out


# speedup vs baseline: 409.2323x; 1.0189x over previous
import jax
import jax.numpy as jnp
from jax.experimental import pallas as pl
from jax.experimental.pallas import tpu as pltpu

TOPK = 2
TS = 1024


def _combine_kernel(pooled_ref, w_ref, b_ref, lora_ref, base_ref, out_ref,
                    idx_ref, buf, sem):
    g = pl.program_id(0)
    total = pl.num_programs(0)
    B, E = pooled_ref.shape[0], w_ref.shape[0]
    S = lora_ref.shape[1]
    nt = S // TS

    @pl.when(g == 0)
    def _gate():
        gate = jax.lax.dot_general(
            pooled_ref[...], w_ref[...], (((1,), (1,)), ((), ())),
            preferred_element_type=jnp.float32,
            precision=jax.lax.Precision.HIGHEST,
        ) + b_ref[...]
        lane = jax.lax.broadcasted_iota(jnp.int32, (B, E), 1)
        rank = jnp.zeros((B, E), jnp.int32)
        for j in range(E):
            gj = gate[:, j:j + 1]
            beats = (gj > gate) | ((gj == gate) & (j < lane))
            rank = rank + beats.astype(jnp.int32)
        for bb in range(B):
            for k in range(TOPK):
                idx_ref[bb, k] = jnp.sum(
                    jnp.where(rank[bb:bb + 1, :] == k, lane[:1], 0))

    def start(gg, slot):
        b = gg // nt
        t = gg % nt
        for k in range(TOPK):
            e = idx_ref[b, k]
            pltpu.make_async_copy(
                lora_ref.at[b, pl.ds(t * TS, TS), e, :],
                buf.at[slot, k],
                sem.at[slot, k],
            ).start()

    p = jax.lax.rem(g, 2)

    @pl.when(g == 0)
    def _first():
        start(g, p)

    @pl.when(g + 1 < total)
    def _prefetch_next():
        start(g + 1, 1 - p)

    def wait(gg, slot):
        b = gg // nt
        t = gg % nt
        for k in range(TOPK):
            e = idx_ref[b, k]
            pltpu.make_async_copy(
                lora_ref.at[b, pl.ds(t * TS, TS), e, :],
                buf.at[slot, k],
                sem.at[slot, k],
            ).wait()

    wait(g, p)
    out_ref[0] = base_ref[0] + (buf[p, 0] + buf[p, 1])


def kernel(x, base_res, lora_results, W, b):
    B, S, D, E = lora_results.shape
    nt = S // TS

    pooled = x[:, -1, :]
    lora_t = lora_results.transpose(0, 1, 3, 2)
    b2 = b.reshape(1, E)

    out = pl.pallas_call(
        _combine_kernel,
        grid=(B * nt,),
        in_specs=[
            pl.BlockSpec((B, D), lambda g: (0, 0)),
            pl.BlockSpec((E, D), lambda g: (0, 0)),
            pl.BlockSpec((1, E), lambda g: (0, 0)),
            pl.BlockSpec(memory_space=pltpu.MemorySpace.HBM),
            pl.BlockSpec((1, TS, D), lambda g: (g // nt, g % nt, 0)),
        ],
        out_specs=pl.BlockSpec((1, TS, D), lambda g: (g // nt, g % nt, 0)),
        out_shape=jax.ShapeDtypeStruct((B, S, D), jnp.float32),
        scratch_shapes=[
            pltpu.SMEM((B, TOPK), jnp.int32),
            pltpu.VMEM((2, TOPK, TS, D), jnp.float32),
            pltpu.SemaphoreType.DMA((2, TOPK)),
        ],
    )(pooled, W, b2, lora_t, base_res)
    return out
